# baseline (speedup 1.0000x reference)
import functools

import jax
import jax.numpy as jnp
from jax import lax
from jax.experimental import pallas as pl
from jax.experimental.pallas import tpu as pltpu
from jax.experimental.pallas import tpu_sc as plsc

NC = 2
NS = 16
NW = NC * NS
IC = 128
NBUF = 8


@functools.lru_cache(maxsize=None)
def _build(B, S, V, D, k=0, nk=1):
    BK = B // nk
    RPW = BK // NW
    NIG = RPW // IC
    NGRP = IC // NBUF
    assert RPW % IC == 0 and IC % NBUF == 0 and NGRP >= 2
    mesh = plsc.VectorSubcoreMesh(core_axis_name="c", subcore_axis_name="s")

    @functools.partial(
        pl.kernel,
        out_type=jax.ShapeDtypeStruct((BK, S, D), jnp.float32),
        mesh=mesh,
        scratch_types=[
            pltpu.VMEM((2, IC, S), jnp.int32),
            pltpu.VMEM((NBUF, S, D), jnp.float32),
            pltpu.SemaphoreType.DMA,
            pltpu.SemaphoreType.DMA,
        ] + [pltpu.SemaphoreType.DMA] * (2 * NBUF),
        compiler_params=pltpu.CompilerParams(use_tc_tiling_on_sc=True),
    )
    def emb_kernel(idx_hbm, emb_hbm, out_hbm, idx_v, bufs, sA, sB, *sems):
        gsems = sems[:NBUF]
        wsems = sems[NBUF:]
        ssems = [sA, sB]
        wid = lax.axis_index("s") * NC + lax.axis_index("c")
        row0 = wid * RPW
        src0 = k * BK + wid * RPW

        def stage(g):
            return pltpu.make_async_copy(
                idx_hbm.at[pl.ds(src0 + g * IC, IC)],
                idx_v.at[g % 2], ssems[g % 2])

        def g_copy(p, r, b):
            return pltpu.make_async_copy(
                emb_hbm.at[idx_v.at[p, r]], bufs.at[b], gsems[b])

        def w_start(row, b):
            pltpu.make_async_copy(
                bufs.at[b], out_hbm.at[row], wsems[b]).start()

        def w_wait(b):
            pltpu.make_async_copy(bufs.at[b], out_hbm.at[0], wsems[b]).wait()

        stage(0).start()
        for g in range(NIG):
            if g + 1 < NIG:
                stage(g + 1).start()
            stage(g).wait()
            p = g % 2
            rbase = row0 + g * IC

            def step(r, b, prefetch):
                pb = (b - 1) % NBUF
                w_wait(pb)
                if prefetch:
                    g_copy(p, r - 1 + NBUF, pb).start()
                g_copy(p, r, b).wait()
                w_start(rbase + r, b)

            for b in range(NBUF):
                g_copy(p, b, b).start()
            g_copy(p, 0, 0).wait()
            w_start(rbase, 0)

            def group(j, carry):
                r0 = 1 + j * NBUF
                for q in range(NBUF):
                    step(r0 + q, (1 + q) % NBUF, prefetch=True)
                return carry

            lax.fori_loop(0, NGRP - 1, group, 0)

            for q in range(NBUF - 1):
                step(IC - NBUF + 1 + q, (1 + q) % NBUF, prefetch=False)
            w_wait((NBUF - 1) % NBUF)

    return emb_kernel


K = 2


def kernel(inputs, embedding):
    B, S = inputs.shape
    V, D = embedding.shape
    idx = inputs.astype(jnp.int32)
    parts = [_build(B, S, V, D, k, K)(idx, embedding) for k in range(K)]
    out = jnp.zeros((B, S, D), jnp.float32)
    for k in range(K):
        out = lax.dynamic_update_slice(out, parts[k], (k * (B // K), 0, 0))
    return out


# speedup vs baseline: 1.4018x; 1.4018x over previous
import functools

import jax
import jax.numpy as jnp
from jax import lax
from jax.experimental import pallas as pl
from jax.experimental.pallas import tpu as pltpu
from jax.experimental.pallas import tpu_sc as plsc

NC = 2
NS = 16
NW = NC * NS
IC = 128
NBUF = 8


@functools.lru_cache(maxsize=None)
def _build(B, S, V, D):
    RPW = B // NW
    NIG = RPW // IC
    NGRP = IC // NBUF
    assert RPW % IC == 0 and IC % NBUF == 0 and NGRP >= 2
    mesh = plsc.VectorSubcoreMesh(core_axis_name="c", subcore_axis_name="s")

    @functools.partial(
        pl.kernel,
        out_type=(),
        mesh=mesh,
        scratch_types=[
            pltpu.VMEM((2, IC, S), jnp.int32),
            pltpu.VMEM((NBUF, S, D), jnp.float32),
            pltpu.SemaphoreType.DMA,
            pltpu.SemaphoreType.DMA,
        ] + [pltpu.SemaphoreType.DMA] * (2 * NBUF),
        compiler_params=pltpu.CompilerParams(use_tc_tiling_on_sc=True),
    )
    def emb_kernel(idx_hbm, emb_hbm, out_hbm, idx_v, bufs, sA, sB, *sems):
        gsems = sems[:NBUF]
        wsems = sems[NBUF:]
        ssems = [sA, sB]
        wid = lax.axis_index("s") * NC + lax.axis_index("c")
        row0 = wid * RPW

        def stage(k):
            return pltpu.make_async_copy(
                idx_hbm.at[pl.ds(row0 + k * IC, IC)],
                idx_v.at[k % 2], ssems[k % 2])

        def g_copy(p, r, b):
            return pltpu.make_async_copy(
                emb_hbm.at[idx_v.at[p, r]], bufs.at[b], gsems[b])

        def w_start(row, b):
            pltpu.make_async_copy(
                bufs.at[b], out_hbm.at[row], wsems[b]).start()

        def w_wait(b):
            pltpu.make_async_copy(bufs.at[b], out_hbm.at[0], wsems[b]).wait()

        stage(0).start()
        for k in range(NIG):
            if k + 1 < NIG:
                stage(k + 1).start()
            stage(k).wait()
            p = k % 2
            rbase = row0 + k * IC

            def step(r, b, prefetch):
                pb = (b - 1) % NBUF
                w_wait(pb)
                if prefetch:
                    g_copy(p, r - 1 + NBUF, pb).start()
                g_copy(p, r, b).wait()
                w_start(rbase + r, b)

            for b in range(NBUF):
                g_copy(p, b, b).start()
            g_copy(p, 0, 0).wait()
            w_start(rbase, 0)

            def group(j, carry):
                r0 = 1 + j * NBUF
                for q in range(NBUF):
                    step(r0 + q, (1 + q) % NBUF, prefetch=True)
                return carry

            lax.fori_loop(0, NGRP - 1, group, 0)

            for q in range(NBUF - 1):
                step(IC - NBUF + 1 + q, (1 + q) % NBUF, prefetch=False)
            w_wait((NBUF - 1) % NBUF)

    return emb_kernel


def kernel(inputs, embedding):
    B, S = inputs.shape
    V, D = embedding.shape
    out = jax.new_ref(jnp.zeros((B, S, D), jnp.float32))
    _build(B, S, V, D)(inputs.astype(jnp.int32), embedding, out)
    return jax.freeze(out)


# speedup vs baseline: 1.7278x; 1.2326x over previous
import functools

import jax
import jax.numpy as jnp
from jax import lax
from jax.experimental import pallas as pl
from jax.experimental.pallas import tpu as pltpu
from jax.experimental.pallas import tpu_sc as plsc

NC = 2
NS = 16
NW = NC * NS
IC = 128
NBUF = 8


@functools.lru_cache(maxsize=None)
def _build(B, S, V, D):
    RPW = B // NW
    NIG = RPW // IC
    NGRP = IC // NBUF
    assert RPW % IC == 0 and IC % NBUF == 0 and NGRP >= 2
    mesh = plsc.VectorSubcoreMesh(core_axis_name="c", subcore_axis_name="s")

    @functools.partial(
        pl.kernel,
        out_type=jax.ShapeDtypeStruct((B, S, D), jnp.float32),
        mesh=mesh,
        scratch_types=[
            pltpu.VMEM((2, IC, S), jnp.int32),
            pltpu.VMEM((NBUF, S, D), jnp.float32),
            pltpu.SemaphoreType.DMA,
            pltpu.SemaphoreType.DMA,
        ] + [pltpu.SemaphoreType.DMA] * (2 * NBUF),
        compiler_params=pltpu.CompilerParams(use_tc_tiling_on_sc=True),
    )
    def emb_kernel(idx_hbm, emb_hbm, out_hbm, idx_v, bufs, sA, sB, *sems):
        gsems = sems[:NBUF]
        wsems = sems[NBUF:]
        ssems = [sA, sB]
        wid = lax.axis_index("s") * NC + lax.axis_index("c")
        row0 = wid * RPW

        def stage(k):
            return pltpu.make_async_copy(
                idx_hbm.at[pl.ds(row0 + k * IC, IC)],
                idx_v.at[k % 2], ssems[k % 2])

        def g_copy(p, r, b):
            return pltpu.make_async_copy(
                emb_hbm.at[idx_v.at[p, r]], bufs.at[b], gsems[b])

        def w_start(row, b):
            pltpu.make_async_copy(
                bufs.at[b], out_hbm.at[row], wsems[b]).start()

        def w_wait(b):
            pltpu.make_async_copy(bufs.at[b], out_hbm.at[0], wsems[b]).wait()

        stage(0).start()
        for k in range(NIG):
            if k + 1 < NIG:
                stage(k + 1).start()
            stage(k).wait()
            p = k % 2
            rbase = row0 + k * IC

            def step(r, b, prefetch):
                pb = (b - 1) % NBUF
                w_wait(pb)
                if prefetch:
                    g_copy(p, r - 1 + NBUF, pb).start()
                g_copy(p, r, b).wait()
                w_start(rbase + r, b)

            for b in range(NBUF):
                g_copy(p, b, b).start()
            g_copy(p, 0, 0).wait()
            w_start(rbase, 0)

            def group(j, carry):
                r0 = 1 + j * NBUF
                for q in range(NBUF):
                    step(r0 + q, (1 + q) % NBUF, prefetch=True)
                return carry

            lax.fori_loop(0, NGRP - 1, group, 0)

            for q in range(NBUF - 1):
                step(IC - NBUF + 1 + q, (1 + q) % NBUF, prefetch=False)
            w_wait((NBUF - 1) % NBUF)

    return emb_kernel


def kernel(inputs, embedding):
    B, S = inputs.shape
    V, D = embedding.shape
    return _build(B, S, V, D)(inputs.astype(jnp.int32), embedding)


# speedup vs baseline: 1.7333x; 1.0032x over previous
import functools

import jax
import jax.numpy as jnp
from jax import lax
from jax.experimental import pallas as pl
from jax.experimental.pallas import tpu as pltpu
from jax.experimental.pallas import tpu_sc as plsc

NC = 2
NS = 16
NW = NC * NS
IC = 128
NBUF = 4
RB = 2


@functools.lru_cache(maxsize=None)
def _build(B, S, V, D):
    RPW = B // NW
    NIG = RPW // IC
    STEPS = IC // RB
    NGRP = STEPS // NBUF
    assert RPW % IC == 0 and STEPS % NBUF == 0 and NGRP >= 2
    mesh = plsc.VectorSubcoreMesh(core_axis_name="c", subcore_axis_name="s")

    @functools.partial(
        pl.kernel,
        out_type=jax.ShapeDtypeStruct((B, S, D), jnp.float32),
        mesh=mesh,
        scratch_types=[
            pltpu.VMEM((2, IC, S), jnp.int32),
            pltpu.VMEM((NBUF, RB, S, D), jnp.float32),
            pltpu.SemaphoreType.DMA,
            pltpu.SemaphoreType.DMA,
        ] + [pltpu.SemaphoreType.DMA] * (2 * NBUF),
        compiler_params=pltpu.CompilerParams(use_tc_tiling_on_sc=True),
    )
    def emb_kernel(idx_hbm, emb_hbm, out_hbm, idx_v, bufs, sA, sB, *sems):
        gsems = sems[:NBUF]
        wsems = sems[NBUF:]
        ssems = [sA, sB]
        wid = lax.axis_index("s") * NC + lax.axis_index("c")
        row0 = wid * RPW

        def stage(k):
            return pltpu.make_async_copy(
                idx_hbm.at[pl.ds(row0 + k * IC, IC)],
                idx_v.at[k % 2], ssems[k % 2])

        def g_copy(p, t, b, j):
            return pltpu.make_async_copy(
                emb_hbm.at[idx_v.at[p, t * RB + j]], bufs.at[b, j],
                gsems[b])

        def w_start(row, b):
            pltpu.make_async_copy(
                bufs.at[b], out_hbm.at[pl.ds(row, RB)], wsems[b]).start()

        def w_wait(b):
            pltpu.make_async_copy(
                bufs.at[b], out_hbm.at[pl.ds(0, RB)], wsems[b]).wait()

        stage(0).start()
        for k in range(NIG):
            if k + 1 < NIG:
                stage(k + 1).start()
            stage(k).wait()
            p = k % 2
            rbase = row0 + k * IC

            def step(t, b, prefetch):
                pb = (b - 1) % NBUF
                w_wait(pb)
                if prefetch:
                    for j in range(RB):
                        g_copy(p, t - 1 + NBUF, pb, j).start()
                for j in range(RB):
                    g_copy(p, t, b, j).wait()
                w_start(rbase + t * RB, b)

            for b in range(NBUF):
                for j in range(RB):
                    g_copy(p, b, b, j).start()
            for j in range(RB):
                g_copy(p, 0, 0, j).wait()
            w_start(rbase, 0)

            def group(gg, carry):
                t0 = 1 + gg * NBUF
                for q in range(NBUF):
                    step(t0 + q, (1 + q) % NBUF, prefetch=True)
                return carry

            lax.fori_loop(0, NGRP - 1, group, 0)

            for q in range(NBUF - 1):
                step(STEPS - NBUF + 1 + q, (1 + q) % NBUF, prefetch=False)
            w_wait((NBUF - 1) % NBUF)

    return emb_kernel


def kernel(inputs, embedding):
    B, S = inputs.shape
    V, D = embedding.shape
    return _build(B, S, V, D)(inputs.astype(jnp.int32), embedding)
